# baseline (speedup 1.0000x reference)
import jax
import jax.numpy as jnp
from jax.experimental import pallas as pl
from jax.experimental.pallas import tpu as pltpu

B, NQ, NK = 1, 2048, 1024
D_MSG, D_FF, D_OUT = 16, 32, 16
TQ = 128

_DN0 = (((0,), (0,)), ((), ()))
_DN1 = (((1,), (0,)), ((), ()))


def _pair_body(qet_ref, qit_ref, kft_ref, kit_ref, wq_ref, wk_ref,
               bt_ref, wbd_ref, b2_ref, o_ref, kx_ref):
    f32, bf16 = jnp.float32, jnp.bfloat16

    @pl.when(pl.program_id(0) == 0)
    def _build_kx():
        kmt = jax.lax.dot_general(wk_ref[...], kit_ref[...], _DN0,
                                  preferred_element_type=f32)
        kx_ref[0:24, :] = kft_ref[...].astype(bf16)
        kx_ref[24:40, :] = kmt.astype(bf16)
        kx_ref[40:57, :] = jnp.ones((17, NK), bf16)
        kx_ref[57:64, :] = jnp.zeros((7, NK), bf16)

    eye24 = (jax.lax.broadcasted_iota(jnp.int32, (24, 24), 0) ==
             jax.lax.broadcasted_iota(jnp.int32, (24, 24), 1)
             ).astype(jnp.bfloat16)
    qf = jax.lax.dot_general(qet_ref[...].astype(bf16), eye24, _DN0,
                             preferred_element_type=f32)
    qm = jax.lax.dot_general(qit_ref[...], wq_ref[...], _DN0,
                             preferred_element_type=f32)
    a = jnp.concatenate(
        [qf, jnp.ones((TQ, D_MSG), f32), qm, jnp.ones((TQ, 1), f32),
         jnp.zeros((TQ, 7), f32)], axis=1)

    qw = (a[:, None, :] * bt_ref[...][None, :, :]).astype(bf16).reshape(
        TQ * D_FF, 64)
    h = jax.lax.dot_general(qw, kx_ref[...], _DN1,
                            preferred_element_type=f32)
    u = (h * 0.5).astype(bf16)
    t = jnp.tanh(u)
    sb = u * t + u
    bias = b2_ref[:, 0:1]
    for g in range(TQ // 8):
        r = jax.lax.dot_general(wbd_ref[...], sb[g * 256:(g + 1) * 256, :],
                                _DN1, preferred_element_type=f32)
        o_ref[g * 8:(g + 1) * 8, :, :] = (r + bias).reshape(8, D_OUT, NK)


def kernel(q_equi, q_inv, k_equi, k_inv, Wq, bq, Wk, bk, W1, b1, W2, b2):
    f32 = jnp.float32
    qet = q_equi.reshape(NQ, 24).T
    qit = q_inv.reshape(NQ, -1).T
    kft = k_equi.reshape(NK, 24).T
    kit = k_inv.reshape(NK, -1).T

    W1q, W1k, W1d = W1[:16], W1[16:32], W1[32:40]
    bias_col = (b1 + W1k.T @ bk + W1q.T @ bq)[:, None]
    bt = jnp.concatenate(
        [jnp.tile(W1d, (3, 1)).T, W1k.T, W1q.T, bias_col,
         jnp.zeros((D_FF, 7), f32)], axis=1)

    wbd = jnp.kron(jnp.eye(8, dtype=f32), W2.T).astype(jnp.bfloat16)
    b2c = jnp.broadcast_to(jnp.tile(b2, (8,))[:, None], (128, 128))

    out_t = pl.pallas_call(
        _pair_body,
        grid=(NQ // TQ,),
        in_specs=[
            pl.BlockSpec((24, TQ), lambda i: (0, i)),
            pl.BlockSpec((64, TQ), lambda i: (0, i)),
            pl.BlockSpec((24, NK), lambda i: (0, 0)),
            pl.BlockSpec((64, NK), lambda i: (0, 0)),
            pl.BlockSpec((64, D_MSG), lambda i: (0, 0)),
            pl.BlockSpec((64, D_MSG), lambda i: (0, 0)),
            pl.BlockSpec((D_FF, 64), lambda i: (0, 0)),
            pl.BlockSpec((128, 256), lambda i: (0, 0)),
            pl.BlockSpec((128, 128), lambda i: (0, 0)),
        ],
        out_specs=pl.BlockSpec((TQ, D_OUT, NK), lambda i: (i, 0, 0)),
        out_shape=jax.ShapeDtypeStruct((NQ, D_OUT, NK), f32),
        scratch_shapes=[pltpu.VMEM((64, NK), jnp.bfloat16)],
    )(qet, qit, kft, kit, Wq, Wk, bt, wbd, b2c)

    return out_t.reshape(B, NQ, D_OUT, NK).transpose(0, 1, 3, 2)
